# baseline (speedup 1.0000x reference)
import functools

import jax
import jax.numpy as jnp
from jax import lax
from jax.experimental import pallas as pl
from jax.experimental.pallas import tpu as pltpu
from jax.experimental.pallas import tpu_sc as plsc

D = 32
B = 16384
L = 50
NC = 2
NS = 16
NW = NC * NS
BW = B // NW
CL = 128
CH = BW // CL

_mesh = plsc.VectorSubcoreMesh(core_axis_name="c", subcore_axis_name="s")


@functools.partial(
    pl.kernel,
    mesh=_mesh,
    out_type=jax.ShapeDtypeStruct((B,), jnp.float32),
    compiler_params=pltpu.CompilerParams(
        needs_layout_passes=False, use_tc_tiling_on_sc=False),
    scratch_types=[
        pltpu.VMEM((L * CH, CL), jnp.int32),
        pltpu.VMEM((CH, CL), jnp.int32),
        pltpu.VMEM((CH, CL), jnp.int32),
        pltpu.VMEM((BW,), jnp.float32),
        pltpu.VMEM((BW, D), jnp.bfloat16),
        pltpu.VMEM((BW, D), jnp.bfloat16),
        pltpu.VMEM((BW, D), jnp.bfloat16),
        pltpu.VMEM((BW, D), jnp.float32),
        pltpu.VMEM((CL, D), jnp.float32),
        pltpu.VMEM((CL, D), jnp.bfloat16),
        pltpu.VMEM((BW,), jnp.float32),
        pltpu.SemaphoreType.DMA,
        pltpu.SemaphoreType.DMA,
    ],
)
def _fpmc_sc(seq_idx_hbm, user_hbm, item_hbm, seqlen_hbm,
             wui_hbm, wiu_hbm, wli_hbm, wil_hbm, out_hbm,
             seq_idx_v, user_v, item_v, seqlen_v,
             vui_v, viu_v, vil_v, acc_v, dummy_v, dummyh_v, out_v,
             sem0, sem1):
    wid = lax.axis_index("s") * NC + lax.axis_index("c")
    base = wid * BW

    pltpu.sync_copy(seq_idx_hbm.at[wid], seq_idx_v)
    pltpu.sync_copy(user_hbm.at[wid], user_v)
    pltpu.sync_copy(item_hbm.at[wid], item_v)
    pltpu.sync_copy(seqlen_hbm.at[wid], seqlen_v)

    zero = jnp.zeros((16,), jnp.float32)
    iota16 = lax.iota(jnp.int32, 16)

    def zbody(i, carry):
        acc_v[i, pl.ds(0, 16)] = zero
        acc_v[i, pl.ds(16, 16)] = zero
        return carry
    lax.fori_loop(0, BW, zbody, 0)

    for c in range(CH):
        dst = pl.ds(c * CL, CL)
        pltpu.async_copy(wui_hbm.at[user_v.at[c]], vui_v.at[dst], sem1)
        pltpu.async_copy(wiu_hbm.at[item_v.at[c]], viu_v.at[dst], sem1)
        pltpu.async_copy(wil_hbm.at[item_v.at[c]], vil_v.at[dst], sem1)

    def fire(t, carry):
        c = lax.rem(t, CH)
        pltpu.async_copy(wli_hbm.at[seq_idx_v.at[t]],
                         acc_v.at[pl.ds(c * CL, CL)], sem0, add=True)
        return carry
    lax.fori_loop(0, L * CH, fire, 0)

    def drain1(t, carry):
        pltpu.make_async_copy(wui_hbm.at[pl.ds(0, CL)], dummyh_v, sem1).wait()
        return carry
    lax.fori_loop(0, 3 * CH, drain1, 0)

    def dot1(g, carry):
        goff = pl.multiple_of(g * 16, 16)
        v = zero
        for j in range(16):
            b = goff + j
            ue, uo = plsc.unpack(vui_v[b, :],
                                 format=plsc.PackFormat.INTERLEAVED)
            ie, io = plsc.unpack(viu_v[b, :],
                                 format=plsc.PackFormat.INTERLEAVED)
            s = jnp.sum(ue * ie + uo * io)
            v = jnp.where(iota16 == j, s, v)
        out_v[pl.ds(goff, 16)] = v
        return carry
    lax.fori_loop(0, BW // 16, dot1, 0)

    def drain0(t, carry):
        pltpu.make_async_copy(wli_hbm.at[pl.ds(0, CL)], dummy_v, sem0).wait()
        return carry
    lax.fori_loop(0, L * CH, drain0, 0)

    cols_e = iota16 * 2
    cols_o = cols_e + 1

    def dot2(g, carry):
        goff = pl.multiple_of(g * 16, 16)
        v = zero
        for j in range(16):
            b = goff + j
            le, lo = plsc.unpack(vil_v[b, :],
                                 format=plsc.PackFormat.INTERLEAVED)
            rows = jnp.full((16,), b, jnp.int32)
            se = plsc.load_gather(acc_v, [rows, cols_e])
            so = plsc.load_gather(acc_v, [rows, cols_o])
            s = jnp.sum(se * le + so * lo)
            v = jnp.where(iota16 == j, s, v)
        sl = seqlen_v[pl.ds(goff, 16)]
        out_v[pl.ds(goff, 16)] = out_v[pl.ds(goff, 16)] + v / sl
        return carry
    lax.fori_loop(0, BW // 16, dot2, 0)

    pltpu.sync_copy(out_v, out_hbm.at[pl.ds(base, BW)])


def kernel(user, item, item_seq, seq_len, W_UI, W_IU, W_LI, W_IL):
    user_r = user.astype(jnp.int32).reshape(NW, CH, CL)
    item_r = item.astype(jnp.int32).reshape(NW, CH, CL)
    seq_r = (item_seq.astype(jnp.int32)
             .reshape(NW, CH, CL, L).transpose(0, 3, 1, 2).reshape(NW, L * CH, CL))
    sl_r = seq_len.reshape(NW, BW)
    wui_h = W_UI.astype(jnp.bfloat16)
    wiu_h = W_IU.astype(jnp.bfloat16)
    wil_h = W_IL.astype(jnp.bfloat16)
    return _fpmc_sc(seq_r, user_r, item_r, sl_r, wui_h, wiu_h, W_LI, wil_h)
